# baseline (speedup 1.0000x reference)
import functools

import jax
import jax.numpy as jnp
from jax import lax
from jax.experimental import pallas as pl
from jax.experimental.pallas import tpu as pltpu
from jax.experimental.pallas import tpu_sc as plsc

N = 10000
D = 128
E = 320000

NC = 2
NS = 16
NW = NC * NS
CB = 8
K = 64
NB = CB * ((E + NW * K * CB - 1) // (NW * K * CB))
TB = NW * NB
E_PAD = TB * K
SINK = N
Q0 = 96
Q1 = (TB - NS * Q0) // NS

TBL = 10112
ZCH = TBL // NS


def _sc_aggregate(with_count):
  mesh = plsc.VectorSubcoreMesh(core_axis_name="c", subcore_axis_name="s",
                                num_cores=NC, num_subcores=NS)
  agg_t = jax.ShapeDtypeStruct((NC, TBL, D), jnp.float32)
  out_type = [agg_t] if with_count else agg_t
  scratch = [
      pltpu.VMEM((CB, K), jnp.int32),
      pltpu.VMEM((CB, K), jnp.int32),
      pltpu.VMEM((K, D), jnp.float32),
      pltpu.VMEM((K, D), jnp.float32),
      pltpu.VMEM_SHARED((TBL, D), jnp.float32),
      pltpu.SemaphoreType.DMA,
      pltpu.SemaphoreType.DMA,
  ]
  if with_count:
    out_type.append(jax.ShapeDtypeStruct((NW, TBL), jnp.float32))
    scratch += [
        pltpu.VMEM((TBL,), jnp.float32),
    ]

  def body(m_hbm, src_hbm, dst_hbm, agg_out, *rest):
    if with_count:
      (cnt_out, src_v, dst_v, rows_a, rows_b, agg_sh,
       sem_a, sem_b, cnt_t) = rest
    else:
      (src_v, dst_v, rows_a, rows_b, agg_sh, sem_a, sem_b) = rest
    rows = (rows_a, rows_b)
    sems = (sem_a, sem_b)
    c = lax.axis_index("c")
    s = lax.axis_index("s")
    wid = s * NC + c
    base = jnp.where(c == 0, s * Q0, NS * Q0 + s * Q1)
    nch = jnp.where(c == 0, Q0 // CB, Q1 // CB)

    zvec = jnp.zeros((16,), jnp.float32)
    @pl.loop(0, 16)
    def zero_zrows(i):
      for j in range(D // 16):
        rows_a[i, pl.ds(j * 16, 16)] = zvec
    if with_count:
      @pl.loop(0, TBL // 16)
      def zero_cnt_t(k):
        cnt_t[pl.ds(k * 16, 16)] = zvec

    @pl.loop(0, ZCH // 16)
    def zero_agg(k):
      pltpu.sync_copy(rows_a.at[pl.ds(0, 16)],
                      agg_sh.at[pl.ds(s * ZCH + k * 16, 16)])
    pltpu.sync_copy(rows_a.at[pl.ds(0, ZCH % 16)],
                    agg_sh.at[pl.ds(s * ZCH + (ZCH // 16) * 16, ZCH % 16)])
    plsc.subcore_barrier()

    ones16 = jnp.ones((16,), jnp.float32)

    @pl.loop(0, nch)
    def chunk(ci):
      off = pl.multiple_of(base + ci * CB, CB)
      pltpu.sync_copy(src_hbm.at[pl.ds(off, CB)], src_v)
      pltpu.sync_copy(dst_hbm.at[pl.ds(off, CB)], dst_v)
      if with_count:
        for bb in range(CB):
          for j in range(K // 16):
            idx = dst_v[bb, pl.ds(j * 16, 16)]
            plsc.addupdate_scatter(cnt_t, [idx], ones16)

      descs = [
          pltpu.async_copy(m_hbm.at[src_v.at[0]], rows[0], sems[0]),
          pltpu.async_copy(m_hbm.at[src_v.at[1]], rows[1], sems[1]),
      ]
      for b in range(CB):
        p = b % 2
        descs[p].wait()
        pltpu.sync_copy(rows[p], agg_sh.at[dst_v.at[b]], add=True)
        if b + 2 < CB:
          descs[p] = pltpu.async_copy(m_hbm.at[src_v.at[b + 2]],
                                      rows[p], sems[p])
    plsc.subcore_barrier()

    pltpu.sync_copy(agg_sh.at[pl.ds(s * ZCH, ZCH)],
                    agg_out.at[c, pl.ds(s * ZCH, ZCH)])
    if with_count:
      pltpu.sync_copy(cnt_t, cnt_out.at[wid])

  return pl.kernel(body, out_type=out_type, mesh=mesh, scratch_types=scratch,
                   compiler_params=pltpu.CompilerParams(needs_layout_passes=False))


_R = 1024
_GRID = (N + _R - 1) // _R


def _tc_pre_body(x_ref, wpre_ref, bpre_ref, wn1_ref, h0_ref, m1_ref):
  h0 = jnp.dot(x_ref[...], wpre_ref[...],
               preferred_element_type=jnp.float32) + bpre_ref[...]
  h0_ref[...] = h0
  m1_ref[...] = jnp.dot(h0, wn1_ref[...], preferred_element_type=jnp.float32)


def _tc_mid_body(hp_ref, agg_ref, cnt_ref, ws_ref, b_ref, wnn_ref,
                 h_ref, mn_ref):
  agg = agg_ref[0] + agg_ref[1]
  cnt = jnp.sum(cnt_ref[...], axis=0)[:, None]
  mean = agg / jnp.maximum(cnt, 1.0)
  h = jnp.maximum(
      jnp.dot(hp_ref[...], ws_ref[...], preferred_element_type=jnp.float32)
      + mean + b_ref[...], 0.0)
  h_ref[...] = h
  mn_ref[...] = jnp.dot(h, wnn_ref[...], preferred_element_type=jnp.float32)


def _tc_post_body(hp_ref, agg_ref, cnt_ref, ws_ref, b_ref, wpost_ref,
                  bpost_ref, out_ref):
  agg = agg_ref[0] + agg_ref[1]
  cnt = jnp.sum(cnt_ref[...], axis=0)[:, None]
  mean = agg / jnp.maximum(cnt, 1.0)
  h = jnp.maximum(
      jnp.dot(hp_ref[...], ws_ref[...], preferred_element_type=jnp.float32)
      + mean + b_ref[...], 0.0)
  out_ref[...] = jnp.sum(h * wpost_ref[...], axis=1,
                         keepdims=True) + bpost_ref[...]


def _row_spec(shape):
  if len(shape) == 2:
    if shape[0] == NW:
      return pl.BlockSpec((NW, _R), lambda i: (0, i))
    return pl.BlockSpec((_R, shape[1]), lambda i: (i, 0))
  return pl.BlockSpec((shape[0], _R, shape[2]), lambda i: (0, i, 0))


def _full_spec(shape):
  return pl.BlockSpec(shape, lambda i: tuple(0 for _ in shape))


def _tc_call(body, in_shapes, out_shapes, row_in, row_out):
  in_specs = [_row_spec(s) if r else _full_spec(s)
              for s, r in zip(in_shapes, row_in)]
  out_specs = [_row_spec(s) if r else _full_spec(s)
               for s, r in zip(out_shapes, row_out)]
  return pl.pallas_call(
      body,
      grid=(_GRID,),
      in_specs=in_specs,
      out_specs=out_specs[0] if len(out_specs) == 1 else out_specs,
      out_shape=[jax.ShapeDtypeStruct(s, jnp.float32) for s in out_shapes]
      if len(out_shapes) > 1 else jax.ShapeDtypeStruct(out_shapes[0],
                                                       jnp.float32),
  )


def kernel(x, edge_index, W_pre, b_pre, W_self1, W_neigh1, b1,
           W_self2, W_neigh2, b2, W_post, b_post):
  src = edge_index[0]
  dst = edge_index[1]

  pad = E_PAD - E
  src_p = jnp.concatenate(
      [src, jnp.zeros((pad,), jnp.int32)]).reshape(TB, K)
  dst_p = jnp.concatenate(
      [dst, jnp.full((pad,), SINK, jnp.int32)]).reshape(TB, K)

  b_pre2 = b_pre.reshape(1, D)
  b12 = b1.reshape(1, D)
  b22 = b2.reshape(1, D)
  wpost_row = W_post.reshape(1, D)
  bpost2 = b_post.reshape(1, 1)

  tc_pre = _tc_call(
      _tc_pre_body,
      [(N, D), (D, D), (1, D), (D, D)],
      [(N, D), (N, D)],
      [True, False, False, False], [True, True])
  h0, m1 = tc_pre(x, W_pre, b_pre2, W_neigh1)

  agg1, cnt = _sc_aggregate(True)(m1, src_p, dst_p)

  tc_mid = _tc_call(
      _tc_mid_body,
      [(N, D), (NC, TBL, D), (NW, TBL), (D, D), (1, D), (D, D)],
      [(N, D), (N, D)],
      [True, True, True, False, False, False], [True, True])
  h1, m2 = tc_mid(h0, agg1, cnt, W_self1, b12, W_neigh2)

  agg2 = _sc_aggregate(False)(m2, src_p, dst_p)

  tc_post = _tc_call(
      _tc_post_body,
      [(N, D), (NC, TBL, D), (NW, TBL), (D, D), (1, D), (1, D), (1, 1)],
      [(N, 1)],
      [True, True, True, False, False, False, False], [True])
  out = tc_post(h1, agg2, cnt, W_self2, b22, wpost_row, bpost2)
  return jnp.squeeze(out, 1)


# speedup vs baseline: 1.1510x; 1.1510x over previous
import functools

import jax
import jax.numpy as jnp
from jax import lax
from jax.experimental import pallas as pl
from jax.experimental.pallas import tpu as pltpu
from jax.experimental.pallas import tpu_sc as plsc

N = 10000
D = 128
E = 320000

NC = 2
NS = 16
NW = NC * NS
CB = 8
K = 64
NB = CB * ((E + NW * K * CB - 1) // (NW * K * CB))
TB = NW * NB
E_PAD = TB * K
SINK = N
Q0 = 224
Q1 = (TB - NS * Q0) // NS

TBL = 10112
ZCH = TBL // NS


def _sc_aggregate(with_count):
  mesh = plsc.VectorSubcoreMesh(core_axis_name="c", subcore_axis_name="s",
                                num_cores=NC, num_subcores=NS)
  agg_t = jax.ShapeDtypeStruct((NC, TBL, D), jnp.float32)
  out_type = [agg_t] if with_count else agg_t
  scratch = [
      pltpu.VMEM((CB, K), jnp.int32),
      pltpu.VMEM((CB, K), jnp.int32),
      pltpu.VMEM((K, D), jnp.float32),
      pltpu.VMEM((K, D), jnp.float32),
      pltpu.VMEM_SHARED((TBL, D), jnp.float32),
      pltpu.SemaphoreType.DMA,
      pltpu.SemaphoreType.DMA,
  ]
  if with_count:
    out_type.append(jax.ShapeDtypeStruct((NW, TBL), jnp.float32))
    scratch += [
        pltpu.VMEM((TBL,), jnp.float32),
    ]

  def body(m_hbm, src_hbm, dst_hbm, agg_out, *rest):
    if with_count:
      (cnt_out, src_v, dst_v, rows_a, rows_b, agg_sh,
       sem_a, sem_b, cnt_t) = rest
    else:
      (src_v, dst_v, rows_a, rows_b, agg_sh, sem_a, sem_b) = rest
    rows = (rows_a, rows_b)
    sems = (sem_a, sem_b)
    c = lax.axis_index("c")
    s = lax.axis_index("s")
    wid = s * NC + c
    base = jnp.where(c == 0, s * Q0, NS * Q0 + s * Q1)
    nch = jnp.where(c == 0, Q0 // CB, Q1 // CB)

    zvec = jnp.zeros((16,), jnp.float32)
    @pl.loop(0, 16)
    def zero_zrows(i):
      for j in range(D // 16):
        rows_a[i, pl.ds(j * 16, 16)] = zvec
    if with_count:
      @pl.loop(0, TBL // 16)
      def zero_cnt_t(k):
        cnt_t[pl.ds(k * 16, 16)] = zvec

    @pl.loop(0, ZCH // 16)
    def zero_agg(k):
      pltpu.sync_copy(rows_a.at[pl.ds(0, 16)],
                      agg_sh.at[pl.ds(s * ZCH + k * 16, 16)])
    pltpu.sync_copy(rows_a.at[pl.ds(0, ZCH % 16)],
                    agg_sh.at[pl.ds(s * ZCH + (ZCH // 16) * 16, ZCH % 16)])
    plsc.subcore_barrier()

    ones16 = jnp.ones((16,), jnp.float32)

    @pl.loop(0, nch)
    def chunk(ci):
      off = pl.multiple_of(base + ci * CB, CB)
      pltpu.sync_copy(src_hbm.at[pl.ds(off, CB)], src_v)
      pltpu.sync_copy(dst_hbm.at[pl.ds(off, CB)], dst_v)
      if with_count:
        for bb in range(CB):
          for j in range(K // 16):
            idx = dst_v[bb, pl.ds(j * 16, 16)]
            plsc.addupdate_scatter(cnt_t, [idx], ones16)

      descs = [
          pltpu.async_copy(m_hbm.at[src_v.at[0]], rows[0], sems[0]),
          pltpu.async_copy(m_hbm.at[src_v.at[1]], rows[1], sems[1]),
      ]
      for b in range(CB):
        p = b % 2
        descs[p].wait()
        pltpu.sync_copy(rows[p], agg_sh.at[dst_v.at[b]], add=True)
        if b + 2 < CB:
          descs[p] = pltpu.async_copy(m_hbm.at[src_v.at[b + 2]],
                                      rows[p], sems[p])
    plsc.subcore_barrier()

    pltpu.sync_copy(agg_sh.at[pl.ds(s * ZCH, ZCH)],
                    agg_out.at[c, pl.ds(s * ZCH, ZCH)])
    if with_count:
      pltpu.sync_copy(cnt_t, cnt_out.at[wid])

  return pl.kernel(body, out_type=out_type, mesh=mesh, scratch_types=scratch,
                   compiler_params=pltpu.CompilerParams(needs_layout_passes=False))


_R = 1024
_GRID = (N + _R - 1) // _R


def _tc_pre_body(x_ref, wpre_ref, bpre_ref, wn1_ref, h0_ref, m1_ref):
  h0 = jnp.dot(x_ref[...], wpre_ref[...],
               preferred_element_type=jnp.float32) + bpre_ref[...]
  h0_ref[...] = h0
  m1_ref[...] = jnp.dot(h0, wn1_ref[...], preferred_element_type=jnp.float32)


def _tc_mid_body(hp_ref, agg_ref, cnt_ref, ws_ref, b_ref, wnn_ref,
                 h_ref, mn_ref):
  agg = agg_ref[0] + agg_ref[1]
  cnt = jnp.sum(cnt_ref[...], axis=0)[:, None]
  mean = agg / jnp.maximum(cnt, 1.0)
  h = jnp.maximum(
      jnp.dot(hp_ref[...], ws_ref[...], preferred_element_type=jnp.float32)
      + mean + b_ref[...], 0.0)
  h_ref[...] = h
  mn_ref[...] = jnp.dot(h, wnn_ref[...], preferred_element_type=jnp.float32)


def _tc_post_body(hp_ref, agg_ref, cnt_ref, ws_ref, b_ref, wpost_ref,
                  bpost_ref, out_ref):
  agg = agg_ref[0] + agg_ref[1]
  cnt = jnp.sum(cnt_ref[...], axis=0)[:, None]
  mean = agg / jnp.maximum(cnt, 1.0)
  h = jnp.maximum(
      jnp.dot(hp_ref[...], ws_ref[...], preferred_element_type=jnp.float32)
      + mean + b_ref[...], 0.0)
  out_ref[...] = jnp.sum(h * wpost_ref[...], axis=1,
                         keepdims=True) + bpost_ref[...]


def _row_spec(shape):
  if len(shape) == 2:
    if shape[0] == NW:
      return pl.BlockSpec((NW, _R), lambda i: (0, i))
    return pl.BlockSpec((_R, shape[1]), lambda i: (i, 0))
  return pl.BlockSpec((shape[0], _R, shape[2]), lambda i: (0, i, 0))


def _full_spec(shape):
  return pl.BlockSpec(shape, lambda i: tuple(0 for _ in shape))


def _tc_call(body, in_shapes, out_shapes, row_in, row_out):
  in_specs = [_row_spec(s) if r else _full_spec(s)
              for s, r in zip(in_shapes, row_in)]
  out_specs = [_row_spec(s) if r else _full_spec(s)
               for s, r in zip(out_shapes, row_out)]
  return pl.pallas_call(
      body,
      grid=(_GRID,),
      in_specs=in_specs,
      out_specs=out_specs[0] if len(out_specs) == 1 else out_specs,
      out_shape=[jax.ShapeDtypeStruct(s, jnp.float32) for s in out_shapes]
      if len(out_shapes) > 1 else jax.ShapeDtypeStruct(out_shapes[0],
                                                       jnp.float32),
  )


def kernel(x, edge_index, W_pre, b_pre, W_self1, W_neigh1, b1,
           W_self2, W_neigh2, b2, W_post, b_post):
  src = edge_index[0]
  dst = edge_index[1]

  pad = E_PAD - E
  src_p = jnp.concatenate(
      [src, jnp.zeros((pad,), jnp.int32)]).reshape(TB, K)
  dst_p = jnp.concatenate(
      [dst, jnp.full((pad,), SINK, jnp.int32)]).reshape(TB, K)

  b_pre2 = b_pre.reshape(1, D)
  b12 = b1.reshape(1, D)
  b22 = b2.reshape(1, D)
  wpost_row = W_post.reshape(1, D)
  bpost2 = b_post.reshape(1, 1)

  tc_pre = _tc_call(
      _tc_pre_body,
      [(N, D), (D, D), (1, D), (D, D)],
      [(N, D), (N, D)],
      [True, False, False, False], [True, True])
  h0, m1 = tc_pre(x, W_pre, b_pre2, W_neigh1)

  agg1, cnt = _sc_aggregate(True)(m1, src_p, dst_p)

  tc_mid = _tc_call(
      _tc_mid_body,
      [(N, D), (NC, TBL, D), (NW, TBL), (D, D), (1, D), (D, D)],
      [(N, D), (N, D)],
      [True, True, True, False, False, False], [True, True])
  h1, m2 = tc_mid(h0, agg1, cnt, W_self1, b12, W_neigh2)

  agg2 = _sc_aggregate(False)(m2, src_p, dst_p)

  tc_post = _tc_call(
      _tc_post_body,
      [(N, D), (NC, TBL, D), (NW, TBL), (D, D), (1, D), (1, D), (1, 1)],
      [(N, 1)],
      [True, True, True, False, False, False, False], [True])
  out = tc_post(h1, agg2, cnt, W_self2, b22, wpost_row, bpost2)
  return jnp.squeeze(out, 1)


# speedup vs baseline: 1.1731x; 1.0192x over previous
import functools

import jax
import jax.numpy as jnp
from jax import lax
from jax.experimental import pallas as pl
from jax.experimental.pallas import tpu as pltpu
from jax.experimental.pallas import tpu_sc as plsc

N = 10000
D = 128
E = 320000

NC = 2
NS = 16
NW = NC * NS
CB = 8
K = 64
NB = CB * ((E + NW * K * CB - 1) // (NW * K * CB))
TB = NW * NB
E_PAD = TB * K
SINK = N
Q0 = 240
Q1 = (TB - NS * Q0) // NS

TBL = 10112
ZCH = TBL // NS


def _sc_aggregate(with_count):
  mesh = plsc.VectorSubcoreMesh(core_axis_name="c", subcore_axis_name="s",
                                num_cores=NC, num_subcores=NS)
  agg_t = jax.ShapeDtypeStruct((NC, TBL, D), jnp.float32)
  out_type = [agg_t] if with_count else agg_t
  scratch = [
      pltpu.VMEM((CB, K), jnp.int32),
      pltpu.VMEM((CB, K), jnp.int32),
      pltpu.VMEM((K, D), jnp.float32),
      pltpu.VMEM((K, D), jnp.float32),
      pltpu.VMEM_SHARED((TBL, D), jnp.float32),
      pltpu.SemaphoreType.DMA,
      pltpu.SemaphoreType.DMA,
  ]
  if with_count:
    out_type.append(jax.ShapeDtypeStruct((NW, TBL), jnp.float32))
    scratch += [
        pltpu.VMEM((TBL,), jnp.float32),
    ]

  def body(m_hbm, src_hbm, dst_hbm, agg_out, *rest):
    if with_count:
      (cnt_out, src_v, dst_v, rows_a, rows_b, agg_sh,
       sem_a, sem_b, cnt_t) = rest
    else:
      (src_v, dst_v, rows_a, rows_b, agg_sh, sem_a, sem_b) = rest
    rows = (rows_a, rows_b)
    sems = (sem_a, sem_b)
    c = lax.axis_index("c")
    s = lax.axis_index("s")
    wid = s * NC + c
    base = jnp.where(c == 0, s * Q0, NS * Q0 + s * Q1)
    nch = jnp.where(c == 0, Q0 // CB, Q1 // CB)

    zvec = jnp.zeros((16,), jnp.float32)
    @pl.loop(0, 16)
    def zero_zrows(i):
      for j in range(D // 16):
        rows_a[i, pl.ds(j * 16, 16)] = zvec
    if with_count:
      @pl.loop(0, TBL // 16)
      def zero_cnt_t(k):
        cnt_t[pl.ds(k * 16, 16)] = zvec

    @pl.loop(0, ZCH // 16)
    def zero_agg(k):
      pltpu.sync_copy(rows_a.at[pl.ds(0, 16)],
                      agg_sh.at[pl.ds(s * ZCH + k * 16, 16)])
    pltpu.sync_copy(rows_a.at[pl.ds(0, ZCH % 16)],
                    agg_sh.at[pl.ds(s * ZCH + (ZCH // 16) * 16, ZCH % 16)])
    plsc.subcore_barrier()

    ones16 = jnp.ones((16,), jnp.float32)

    @pl.loop(0, nch)
    def chunk(ci):
      off = pl.multiple_of(base + ci * CB, CB)
      pltpu.sync_copy(src_hbm.at[pl.ds(off, CB)], src_v)
      pltpu.sync_copy(dst_hbm.at[pl.ds(off, CB)], dst_v)
      if with_count:
        for bb in range(CB):
          for j in range(K // 16):
            idx = dst_v[bb, pl.ds(j * 16, 16)]
            plsc.addupdate_scatter(cnt_t, [idx], ones16)

      descs = [
          pltpu.async_copy(m_hbm.at[src_v.at[0]], rows[0], sems[0]),
          pltpu.async_copy(m_hbm.at[src_v.at[1]], rows[1], sems[1]),
      ]
      for b in range(CB):
        p = b % 2
        descs[p].wait()
        pltpu.sync_copy(rows[p], agg_sh.at[dst_v.at[b]], add=True)
        if b + 2 < CB:
          descs[p] = pltpu.async_copy(m_hbm.at[src_v.at[b + 2]],
                                      rows[p], sems[p])
    plsc.subcore_barrier()

    pltpu.sync_copy(agg_sh.at[pl.ds(s * ZCH, ZCH)],
                    agg_out.at[c, pl.ds(s * ZCH, ZCH)])
    if with_count:
      pltpu.sync_copy(cnt_t, cnt_out.at[wid])

  return pl.kernel(body, out_type=out_type, mesh=mesh, scratch_types=scratch,
                   compiler_params=pltpu.CompilerParams(needs_layout_passes=False))


_R = 1024
_GRID = (N + _R - 1) // _R


def _tc_pre_body(x_ref, wpre_ref, bpre_ref, wn1_ref, h0_ref, m1_ref):
  h0 = jnp.dot(x_ref[...], wpre_ref[...],
               preferred_element_type=jnp.float32) + bpre_ref[...]
  h0_ref[...] = h0
  m1_ref[...] = jnp.dot(h0, wn1_ref[...], preferred_element_type=jnp.float32)


def _tc_mid_body(hp_ref, agg_ref, cnt_ref, ws_ref, b_ref, wnn_ref,
                 h_ref, mn_ref):
  agg = agg_ref[0] + agg_ref[1]
  cnt = jnp.sum(cnt_ref[...], axis=0)[:, None]
  mean = agg / jnp.maximum(cnt, 1.0)
  h = jnp.maximum(
      jnp.dot(hp_ref[...], ws_ref[...], preferred_element_type=jnp.float32)
      + mean + b_ref[...], 0.0)
  h_ref[...] = h
  mn_ref[...] = jnp.dot(h, wnn_ref[...], preferred_element_type=jnp.float32)


def _tc_post_body(hp_ref, agg_ref, cnt_ref, ws_ref, b_ref, wpost_ref,
                  bpost_ref, out_ref):
  agg = agg_ref[0] + agg_ref[1]
  cnt = jnp.sum(cnt_ref[...], axis=0)[:, None]
  mean = agg / jnp.maximum(cnt, 1.0)
  h = jnp.maximum(
      jnp.dot(hp_ref[...], ws_ref[...], preferred_element_type=jnp.float32)
      + mean + b_ref[...], 0.0)
  out_ref[...] = jnp.sum(h * wpost_ref[...], axis=1,
                         keepdims=True) + bpost_ref[...]


def _row_spec(shape):
  if len(shape) == 2:
    if shape[0] == NW:
      return pl.BlockSpec((NW, _R), lambda i: (0, i))
    return pl.BlockSpec((_R, shape[1]), lambda i: (i, 0))
  return pl.BlockSpec((shape[0], _R, shape[2]), lambda i: (0, i, 0))


def _full_spec(shape):
  return pl.BlockSpec(shape, lambda i: tuple(0 for _ in shape))


def _tc_call(body, in_shapes, out_shapes, row_in, row_out):
  in_specs = [_row_spec(s) if r else _full_spec(s)
              for s, r in zip(in_shapes, row_in)]
  out_specs = [_row_spec(s) if r else _full_spec(s)
               for s, r in zip(out_shapes, row_out)]
  return pl.pallas_call(
      body,
      grid=(_GRID,),
      in_specs=in_specs,
      out_specs=out_specs[0] if len(out_specs) == 1 else out_specs,
      out_shape=[jax.ShapeDtypeStruct(s, jnp.float32) for s in out_shapes]
      if len(out_shapes) > 1 else jax.ShapeDtypeStruct(out_shapes[0],
                                                       jnp.float32),
  )


def kernel(x, edge_index, W_pre, b_pre, W_self1, W_neigh1, b1,
           W_self2, W_neigh2, b2, W_post, b_post):
  src = edge_index[0]
  dst = edge_index[1]

  pad = E_PAD - E
  src_p = jnp.concatenate(
      [src, jnp.zeros((pad,), jnp.int32)]).reshape(TB, K)
  dst_p = jnp.concatenate(
      [dst, jnp.full((pad,), SINK, jnp.int32)]).reshape(TB, K)

  b_pre2 = b_pre.reshape(1, D)
  b12 = b1.reshape(1, D)
  b22 = b2.reshape(1, D)
  wpost_row = W_post.reshape(1, D)
  bpost2 = b_post.reshape(1, 1)

  tc_pre = _tc_call(
      _tc_pre_body,
      [(N, D), (D, D), (1, D), (D, D)],
      [(N, D), (N, D)],
      [True, False, False, False], [True, True])
  h0, m1 = tc_pre(x, W_pre, b_pre2, W_neigh1)

  agg1, cnt = _sc_aggregate(True)(m1, src_p, dst_p)

  tc_mid = _tc_call(
      _tc_mid_body,
      [(N, D), (NC, TBL, D), (NW, TBL), (D, D), (1, D), (D, D)],
      [(N, D), (N, D)],
      [True, True, True, False, False, False], [True, True])
  h1, m2 = tc_mid(h0, agg1, cnt, W_self1, b12, W_neigh2)

  agg2 = _sc_aggregate(False)(m2, src_p, dst_p)

  tc_post = _tc_call(
      _tc_post_body,
      [(N, D), (NC, TBL, D), (NW, TBL), (D, D), (1, D), (1, D), (1, 1)],
      [(N, 1)],
      [True, True, True, False, False, False, False], [True])
  out = tc_post(h1, agg2, cnt, W_self2, b22, wpost_row, bpost2)
  return jnp.squeeze(out, 1)


# speedup vs baseline: 1.1955x; 1.0191x over previous
import functools

import jax
import jax.numpy as jnp
from jax import lax
from jax.experimental import pallas as pl
from jax.experimental.pallas import tpu as pltpu
from jax.experimental.pallas import tpu_sc as plsc

N = 10000
D = 128
E = 320000

NC = 2
NS = 16
NW = NC * NS
CB = 8
K = 64
NB = CB * ((E + NW * K * CB - 1) // (NW * K * CB))
TB = NW * NB
E_PAD = TB * K
SINK = N
Q0 = 256
Q1 = (TB - NS * Q0) // NS

TBL = 10112
ZCH = TBL // NS


def _sc_aggregate(with_count):
  mesh = plsc.VectorSubcoreMesh(core_axis_name="c", subcore_axis_name="s",
                                num_cores=NC, num_subcores=NS)
  agg_t = jax.ShapeDtypeStruct((NC, TBL, D), jnp.float32)
  out_type = [agg_t] if with_count else agg_t
  scratch = [
      pltpu.VMEM((CB, K), jnp.int32),
      pltpu.VMEM((CB, K), jnp.int32),
      pltpu.VMEM((K, D), jnp.float32),
      pltpu.VMEM((K, D), jnp.float32),
      pltpu.VMEM_SHARED((TBL, D), jnp.float32),
      pltpu.SemaphoreType.DMA,
      pltpu.SemaphoreType.DMA,
  ]
  if with_count:
    out_type.append(jax.ShapeDtypeStruct((NW, TBL), jnp.float32))
    scratch += [
        pltpu.VMEM((TBL,), jnp.float32),
    ]

  def body(m_hbm, src_hbm, dst_hbm, agg_out, *rest):
    if with_count:
      (cnt_out, src_v, dst_v, rows_a, rows_b, agg_sh,
       sem_a, sem_b, cnt_t) = rest
    else:
      (src_v, dst_v, rows_a, rows_b, agg_sh, sem_a, sem_b) = rest
    rows = (rows_a, rows_b)
    sems = (sem_a, sem_b)
    c = lax.axis_index("c")
    s = lax.axis_index("s")
    wid = s * NC + c
    base = jnp.where(c == 0, s * Q0, NS * Q0 + s * Q1)
    nch = jnp.where(c == 0, Q0 // CB, Q1 // CB)

    zvec = jnp.zeros((16,), jnp.float32)
    @pl.loop(0, 16)
    def zero_zrows(i):
      for j in range(D // 16):
        rows_a[i, pl.ds(j * 16, 16)] = zvec
    if with_count:
      @pl.loop(0, TBL // 16)
      def zero_cnt_t(k):
        cnt_t[pl.ds(k * 16, 16)] = zvec

    @pl.loop(0, ZCH // 16)
    def zero_agg(k):
      pltpu.sync_copy(rows_a.at[pl.ds(0, 16)],
                      agg_sh.at[pl.ds(s * ZCH + k * 16, 16)])
    pltpu.sync_copy(rows_a.at[pl.ds(0, ZCH % 16)],
                    agg_sh.at[pl.ds(s * ZCH + (ZCH // 16) * 16, ZCH % 16)])
    plsc.subcore_barrier()

    ones16 = jnp.ones((16,), jnp.float32)

    @pl.loop(0, nch)
    def chunk(ci):
      off = pl.multiple_of(base + ci * CB, CB)
      pltpu.sync_copy(src_hbm.at[pl.ds(off, CB)], src_v)
      pltpu.sync_copy(dst_hbm.at[pl.ds(off, CB)], dst_v)
      if with_count:
        for bb in range(CB):
          for j in range(K // 16):
            idx = dst_v[bb, pl.ds(j * 16, 16)]
            plsc.addupdate_scatter(cnt_t, [idx], ones16)

      descs = [
          pltpu.async_copy(m_hbm.at[src_v.at[0]], rows[0], sems[0]),
          pltpu.async_copy(m_hbm.at[src_v.at[1]], rows[1], sems[1]),
      ]
      for b in range(CB):
        p = b % 2
        descs[p].wait()
        pltpu.sync_copy(rows[p], agg_sh.at[dst_v.at[b]], add=True)
        if b + 2 < CB:
          descs[p] = pltpu.async_copy(m_hbm.at[src_v.at[b + 2]],
                                      rows[p], sems[p])
    plsc.subcore_barrier()

    pltpu.sync_copy(agg_sh.at[pl.ds(s * ZCH, ZCH)],
                    agg_out.at[c, pl.ds(s * ZCH, ZCH)])
    if with_count:
      pltpu.sync_copy(cnt_t, cnt_out.at[wid])

  return pl.kernel(body, out_type=out_type, mesh=mesh, scratch_types=scratch,
                   compiler_params=pltpu.CompilerParams(needs_layout_passes=False))


_R = 1024
_GRID = (N + _R - 1) // _R


def _tc_pre_body(x_ref, wpre_ref, bpre_ref, wn1_ref, h0_ref, m1_ref):
  h0 = jnp.dot(x_ref[...], wpre_ref[...],
               preferred_element_type=jnp.float32) + bpre_ref[...]
  h0_ref[...] = h0
  m1_ref[...] = jnp.dot(h0, wn1_ref[...], preferred_element_type=jnp.float32)


def _tc_mid_body(hp_ref, agg_ref, cnt_ref, ws_ref, b_ref, wnn_ref,
                 h_ref, mn_ref):
  agg = agg_ref[0] + agg_ref[1]
  cnt = jnp.sum(cnt_ref[...], axis=0)[:, None]
  mean = agg / jnp.maximum(cnt, 1.0)
  h = jnp.maximum(
      jnp.dot(hp_ref[...], ws_ref[...], preferred_element_type=jnp.float32)
      + mean + b_ref[...], 0.0)
  h_ref[...] = h
  mn_ref[...] = jnp.dot(h, wnn_ref[...], preferred_element_type=jnp.float32)


def _tc_post_body(hp_ref, agg_ref, cnt_ref, ws_ref, b_ref, wpost_ref,
                  bpost_ref, out_ref):
  agg = agg_ref[0] + agg_ref[1]
  cnt = jnp.sum(cnt_ref[...], axis=0)[:, None]
  mean = agg / jnp.maximum(cnt, 1.0)
  h = jnp.maximum(
      jnp.dot(hp_ref[...], ws_ref[...], preferred_element_type=jnp.float32)
      + mean + b_ref[...], 0.0)
  out_ref[...] = jnp.sum(h * wpost_ref[...], axis=1,
                         keepdims=True) + bpost_ref[...]


def _row_spec(shape):
  if len(shape) == 2:
    if shape[0] == NW:
      return pl.BlockSpec((NW, _R), lambda i: (0, i))
    return pl.BlockSpec((_R, shape[1]), lambda i: (i, 0))
  return pl.BlockSpec((shape[0], _R, shape[2]), lambda i: (0, i, 0))


def _full_spec(shape):
  return pl.BlockSpec(shape, lambda i: tuple(0 for _ in shape))


def _tc_call(body, in_shapes, out_shapes, row_in, row_out):
  in_specs = [_row_spec(s) if r else _full_spec(s)
              for s, r in zip(in_shapes, row_in)]
  out_specs = [_row_spec(s) if r else _full_spec(s)
               for s, r in zip(out_shapes, row_out)]
  return pl.pallas_call(
      body,
      grid=(_GRID,),
      in_specs=in_specs,
      out_specs=out_specs[0] if len(out_specs) == 1 else out_specs,
      out_shape=[jax.ShapeDtypeStruct(s, jnp.float32) for s in out_shapes]
      if len(out_shapes) > 1 else jax.ShapeDtypeStruct(out_shapes[0],
                                                       jnp.float32),
  )


def kernel(x, edge_index, W_pre, b_pre, W_self1, W_neigh1, b1,
           W_self2, W_neigh2, b2, W_post, b_post):
  src = edge_index[0]
  dst = edge_index[1]

  pad = E_PAD - E
  src_p = jnp.concatenate(
      [src, jnp.zeros((pad,), jnp.int32)]).reshape(TB, K)
  dst_p = jnp.concatenate(
      [dst, jnp.full((pad,), SINK, jnp.int32)]).reshape(TB, K)

  b_pre2 = b_pre.reshape(1, D)
  b12 = b1.reshape(1, D)
  b22 = b2.reshape(1, D)
  wpost_row = W_post.reshape(1, D)
  bpost2 = b_post.reshape(1, 1)

  tc_pre = _tc_call(
      _tc_pre_body,
      [(N, D), (D, D), (1, D), (D, D)],
      [(N, D), (N, D)],
      [True, False, False, False], [True, True])
  h0, m1 = tc_pre(x, W_pre, b_pre2, W_neigh1)

  agg1, cnt = _sc_aggregate(True)(m1, src_p, dst_p)

  tc_mid = _tc_call(
      _tc_mid_body,
      [(N, D), (NC, TBL, D), (NW, TBL), (D, D), (1, D), (D, D)],
      [(N, D), (N, D)],
      [True, True, True, False, False, False], [True, True])
  h1, m2 = tc_mid(h0, agg1, cnt, W_self1, b12, W_neigh2)

  agg2 = _sc_aggregate(False)(m2, src_p, dst_p)

  tc_post = _tc_call(
      _tc_post_body,
      [(N, D), (NC, TBL, D), (NW, TBL), (D, D), (1, D), (1, D), (1, 1)],
      [(N, 1)],
      [True, True, True, False, False, False, False], [True])
  out = tc_post(h1, agg2, cnt, W_self2, b22, wpost_row, bpost2)
  return jnp.squeeze(out, 1)


# speedup vs baseline: 1.1961x; 1.0005x over previous
import functools

import jax
import jax.numpy as jnp
from jax import lax
from jax.experimental import pallas as pl
from jax.experimental.pallas import tpu as pltpu
from jax.experimental.pallas import tpu_sc as plsc

N = 10000
D = 128
E = 320000

NC = 2
NS = 16
NW = NC * NS
CB = 8
K = 64
NB = CB * ((E + NW * K * CB - 1) // (NW * K * CB))
TB = NW * NB
E_PAD = TB * K
SINK = N
Q0 = 288
Q1 = (TB - NS * Q0) // NS

TBL = 10112
ZCH = TBL // NS


def _sc_aggregate(with_count):
  mesh = plsc.VectorSubcoreMesh(core_axis_name="c", subcore_axis_name="s",
                                num_cores=NC, num_subcores=NS)
  agg_t = jax.ShapeDtypeStruct((NC, TBL, D), jnp.float32)
  out_type = [agg_t] if with_count else agg_t
  scratch = [
      pltpu.VMEM((CB, K), jnp.int32),
      pltpu.VMEM((CB, K), jnp.int32),
      pltpu.VMEM((K, D), jnp.float32),
      pltpu.VMEM((K, D), jnp.float32),
      pltpu.VMEM_SHARED((TBL, D), jnp.float32),
      pltpu.SemaphoreType.DMA,
      pltpu.SemaphoreType.DMA,
  ]
  if with_count:
    out_type.append(jax.ShapeDtypeStruct((NW, TBL), jnp.float32))
    scratch += [
        pltpu.VMEM((TBL,), jnp.float32),
    ]

  def body(m_hbm, src_hbm, dst_hbm, agg_out, *rest):
    if with_count:
      (cnt_out, src_v, dst_v, rows_a, rows_b, agg_sh,
       sem_a, sem_b, cnt_t) = rest
    else:
      (src_v, dst_v, rows_a, rows_b, agg_sh, sem_a, sem_b) = rest
    rows = (rows_a, rows_b)
    sems = (sem_a, sem_b)
    c = lax.axis_index("c")
    s = lax.axis_index("s")
    wid = s * NC + c
    base = jnp.where(c == 0, s * Q0, NS * Q0 + s * Q1)
    nch = jnp.where(c == 0, Q0 // CB, Q1 // CB)

    zvec = jnp.zeros((16,), jnp.float32)
    @pl.loop(0, 16)
    def zero_zrows(i):
      for j in range(D // 16):
        rows_a[i, pl.ds(j * 16, 16)] = zvec
    if with_count:
      @pl.loop(0, TBL // 16)
      def zero_cnt_t(k):
        cnt_t[pl.ds(k * 16, 16)] = zvec

    @pl.loop(0, ZCH // 16)
    def zero_agg(k):
      pltpu.sync_copy(rows_a.at[pl.ds(0, 16)],
                      agg_sh.at[pl.ds(s * ZCH + k * 16, 16)])
    pltpu.sync_copy(rows_a.at[pl.ds(0, ZCH % 16)],
                    agg_sh.at[pl.ds(s * ZCH + (ZCH // 16) * 16, ZCH % 16)])
    plsc.subcore_barrier()

    ones16 = jnp.ones((16,), jnp.float32)

    @pl.loop(0, nch)
    def chunk(ci):
      off = pl.multiple_of(base + ci * CB, CB)
      pltpu.sync_copy(src_hbm.at[pl.ds(off, CB)], src_v)
      pltpu.sync_copy(dst_hbm.at[pl.ds(off, CB)], dst_v)
      if with_count:
        for bb in range(CB):
          for j in range(K // 16):
            idx = dst_v[bb, pl.ds(j * 16, 16)]
            plsc.addupdate_scatter(cnt_t, [idx], ones16)

      descs = [
          pltpu.async_copy(m_hbm.at[src_v.at[0]], rows[0], sems[0]),
          pltpu.async_copy(m_hbm.at[src_v.at[1]], rows[1], sems[1]),
      ]
      for b in range(CB):
        p = b % 2
        descs[p].wait()
        pltpu.sync_copy(rows[p], agg_sh.at[dst_v.at[b]], add=True)
        if b + 2 < CB:
          descs[p] = pltpu.async_copy(m_hbm.at[src_v.at[b + 2]],
                                      rows[p], sems[p])
    plsc.subcore_barrier()

    pltpu.sync_copy(agg_sh.at[pl.ds(s * ZCH, ZCH)],
                    agg_out.at[c, pl.ds(s * ZCH, ZCH)])
    if with_count:
      pltpu.sync_copy(cnt_t, cnt_out.at[wid])

  return pl.kernel(body, out_type=out_type, mesh=mesh, scratch_types=scratch,
                   compiler_params=pltpu.CompilerParams(needs_layout_passes=False))


_R = 1024
_GRID = (N + _R - 1) // _R


def _tc_pre_body(x_ref, wpre_ref, bpre_ref, wn1_ref, h0_ref, m1_ref):
  h0 = jnp.dot(x_ref[...], wpre_ref[...],
               preferred_element_type=jnp.float32) + bpre_ref[...]
  h0_ref[...] = h0
  m1_ref[...] = jnp.dot(h0, wn1_ref[...], preferred_element_type=jnp.float32)


def _tc_mid_body(hp_ref, agg_ref, cnt_ref, ws_ref, b_ref, wnn_ref,
                 h_ref, mn_ref):
  agg = agg_ref[0] + agg_ref[1]
  cnt = jnp.sum(cnt_ref[...], axis=0)[:, None]
  mean = agg / jnp.maximum(cnt, 1.0)
  h = jnp.maximum(
      jnp.dot(hp_ref[...], ws_ref[...], preferred_element_type=jnp.float32)
      + mean + b_ref[...], 0.0)
  h_ref[...] = h
  mn_ref[...] = jnp.dot(h, wnn_ref[...], preferred_element_type=jnp.float32)


def _tc_post_body(hp_ref, agg_ref, cnt_ref, ws_ref, b_ref, wpost_ref,
                  bpost_ref, out_ref):
  agg = agg_ref[0] + agg_ref[1]
  cnt = jnp.sum(cnt_ref[...], axis=0)[:, None]
  mean = agg / jnp.maximum(cnt, 1.0)
  h = jnp.maximum(
      jnp.dot(hp_ref[...], ws_ref[...], preferred_element_type=jnp.float32)
      + mean + b_ref[...], 0.0)
  out_ref[...] = jnp.sum(h * wpost_ref[...], axis=1,
                         keepdims=True) + bpost_ref[...]


def _row_spec(shape):
  if len(shape) == 2:
    if shape[0] == NW:
      return pl.BlockSpec((NW, _R), lambda i: (0, i))
    return pl.BlockSpec((_R, shape[1]), lambda i: (i, 0))
  return pl.BlockSpec((shape[0], _R, shape[2]), lambda i: (0, i, 0))


def _full_spec(shape):
  return pl.BlockSpec(shape, lambda i: tuple(0 for _ in shape))


def _tc_call(body, in_shapes, out_shapes, row_in, row_out):
  in_specs = [_row_spec(s) if r else _full_spec(s)
              for s, r in zip(in_shapes, row_in)]
  out_specs = [_row_spec(s) if r else _full_spec(s)
               for s, r in zip(out_shapes, row_out)]
  return pl.pallas_call(
      body,
      grid=(_GRID,),
      in_specs=in_specs,
      out_specs=out_specs[0] if len(out_specs) == 1 else out_specs,
      out_shape=[jax.ShapeDtypeStruct(s, jnp.float32) for s in out_shapes]
      if len(out_shapes) > 1 else jax.ShapeDtypeStruct(out_shapes[0],
                                                       jnp.float32),
  )


def kernel(x, edge_index, W_pre, b_pre, W_self1, W_neigh1, b1,
           W_self2, W_neigh2, b2, W_post, b_post):
  src = edge_index[0]
  dst = edge_index[1]

  pad = E_PAD - E
  src_p = jnp.concatenate(
      [src, jnp.zeros((pad,), jnp.int32)]).reshape(TB, K)
  dst_p = jnp.concatenate(
      [dst, jnp.full((pad,), SINK, jnp.int32)]).reshape(TB, K)

  b_pre2 = b_pre.reshape(1, D)
  b12 = b1.reshape(1, D)
  b22 = b2.reshape(1, D)
  wpost_row = W_post.reshape(1, D)
  bpost2 = b_post.reshape(1, 1)

  tc_pre = _tc_call(
      _tc_pre_body,
      [(N, D), (D, D), (1, D), (D, D)],
      [(N, D), (N, D)],
      [True, False, False, False], [True, True])
  h0, m1 = tc_pre(x, W_pre, b_pre2, W_neigh1)

  agg1, cnt = _sc_aggregate(True)(m1, src_p, dst_p)

  tc_mid = _tc_call(
      _tc_mid_body,
      [(N, D), (NC, TBL, D), (NW, TBL), (D, D), (1, D), (D, D)],
      [(N, D), (N, D)],
      [True, True, True, False, False, False], [True, True])
  h1, m2 = tc_mid(h0, agg1, cnt, W_self1, b12, W_neigh2)

  agg2 = _sc_aggregate(False)(m2, src_p, dst_p)

  tc_post = _tc_call(
      _tc_post_body,
      [(N, D), (NC, TBL, D), (NW, TBL), (D, D), (1, D), (1, D), (1, 1)],
      [(N, 1)],
      [True, True, True, False, False, False, False], [True])
  out = tc_post(h1, agg2, cnt, W_self2, b22, wpost_row, bpost2)
  return jnp.squeeze(out, 1)


# speedup vs baseline: 1.1968x; 1.0006x over previous
import functools

import jax
import jax.numpy as jnp
from jax import lax
from jax.experimental import pallas as pl
from jax.experimental.pallas import tpu as pltpu
from jax.experimental.pallas import tpu_sc as plsc

N = 10000
D = 128
E = 320000

NC = 2
NS = 16
NW = NC * NS
CB = 16
K = 64
NB = CB * ((E + NW * K * CB - 1) // (NW * K * CB))
TB = NW * NB
E_PAD = TB * K
SINK = N
Q0 = 256
Q1 = (TB - NS * Q0) // NS

TBL = 10112
ZCH = TBL // NS


def _sc_aggregate(with_count):
  mesh = plsc.VectorSubcoreMesh(core_axis_name="c", subcore_axis_name="s",
                                num_cores=NC, num_subcores=NS)
  agg_t = jax.ShapeDtypeStruct((NC, TBL, D), jnp.float32)
  out_type = [agg_t] if with_count else agg_t
  scratch = [
      pltpu.VMEM((CB, K), jnp.int32),
      pltpu.VMEM((CB, K), jnp.int32),
      pltpu.VMEM((K, D), jnp.float32),
      pltpu.VMEM((K, D), jnp.float32),
      pltpu.VMEM_SHARED((TBL, D), jnp.float32),
      pltpu.SemaphoreType.DMA,
      pltpu.SemaphoreType.DMA,
  ]
  if with_count:
    out_type.append(jax.ShapeDtypeStruct((NW, TBL), jnp.float32))
    scratch += [
        pltpu.VMEM((TBL,), jnp.float32),
    ]

  def body(m_hbm, src_hbm, dst_hbm, agg_out, *rest):
    if with_count:
      (cnt_out, src_v, dst_v, rows_a, rows_b, agg_sh,
       sem_a, sem_b, cnt_t) = rest
    else:
      (src_v, dst_v, rows_a, rows_b, agg_sh, sem_a, sem_b) = rest
    rows = (rows_a, rows_b)
    sems = (sem_a, sem_b)
    c = lax.axis_index("c")
    s = lax.axis_index("s")
    wid = s * NC + c
    base = jnp.where(c == 0, s * Q0, NS * Q0 + s * Q1)
    nch = jnp.where(c == 0, Q0 // CB, Q1 // CB)

    zvec = jnp.zeros((16,), jnp.float32)
    @pl.loop(0, 16)
    def zero_zrows(i):
      for j in range(D // 16):
        rows_a[i, pl.ds(j * 16, 16)] = zvec
    if with_count:
      @pl.loop(0, TBL // 16)
      def zero_cnt_t(k):
        cnt_t[pl.ds(k * 16, 16)] = zvec

    @pl.loop(0, ZCH // 16)
    def zero_agg(k):
      pltpu.sync_copy(rows_a.at[pl.ds(0, 16)],
                      agg_sh.at[pl.ds(s * ZCH + k * 16, 16)])
    pltpu.sync_copy(rows_a.at[pl.ds(0, ZCH % 16)],
                    agg_sh.at[pl.ds(s * ZCH + (ZCH // 16) * 16, ZCH % 16)])
    plsc.subcore_barrier()

    ones16 = jnp.ones((16,), jnp.float32)

    @pl.loop(0, nch)
    def chunk(ci):
      off = pl.multiple_of(base + ci * CB, CB)
      pltpu.sync_copy(src_hbm.at[pl.ds(off, CB)], src_v)
      pltpu.sync_copy(dst_hbm.at[pl.ds(off, CB)], dst_v)
      if with_count:
        for bb in range(CB):
          for j in range(K // 16):
            idx = dst_v[bb, pl.ds(j * 16, 16)]
            plsc.addupdate_scatter(cnt_t, [idx], ones16)

      descs = [
          pltpu.async_copy(m_hbm.at[src_v.at[0]], rows[0], sems[0]),
          pltpu.async_copy(m_hbm.at[src_v.at[1]], rows[1], sems[1]),
      ]
      for b in range(CB):
        p = b % 2
        descs[p].wait()
        pltpu.sync_copy(rows[p], agg_sh.at[dst_v.at[b]], add=True)
        if b + 2 < CB:
          descs[p] = pltpu.async_copy(m_hbm.at[src_v.at[b + 2]],
                                      rows[p], sems[p])
    plsc.subcore_barrier()

    pltpu.sync_copy(agg_sh.at[pl.ds(s * ZCH, ZCH)],
                    agg_out.at[c, pl.ds(s * ZCH, ZCH)])
    if with_count:
      pltpu.sync_copy(cnt_t, cnt_out.at[wid])

  return pl.kernel(body, out_type=out_type, mesh=mesh, scratch_types=scratch,
                   compiler_params=pltpu.CompilerParams(needs_layout_passes=False))


_R = 1024
_GRID = (N + _R - 1) // _R


def _tc_pre_body(x_ref, wpre_ref, bpre_ref, wn1_ref, h0_ref, m1_ref):
  h0 = jnp.dot(x_ref[...], wpre_ref[...],
               preferred_element_type=jnp.float32) + bpre_ref[...]
  h0_ref[...] = h0
  m1_ref[...] = jnp.dot(h0, wn1_ref[...], preferred_element_type=jnp.float32)


def _tc_mid_body(hp_ref, agg_ref, cnt_ref, ws_ref, b_ref, wnn_ref,
                 h_ref, mn_ref):
  agg = agg_ref[0] + agg_ref[1]
  cnt = jnp.sum(cnt_ref[...], axis=0)[:, None]
  mean = agg / jnp.maximum(cnt, 1.0)
  h = jnp.maximum(
      jnp.dot(hp_ref[...], ws_ref[...], preferred_element_type=jnp.float32)
      + mean + b_ref[...], 0.0)
  h_ref[...] = h
  mn_ref[...] = jnp.dot(h, wnn_ref[...], preferred_element_type=jnp.float32)


def _tc_post_body(hp_ref, agg_ref, cnt_ref, ws_ref, b_ref, wpost_ref,
                  bpost_ref, out_ref):
  agg = agg_ref[0] + agg_ref[1]
  cnt = jnp.sum(cnt_ref[...], axis=0)[:, None]
  mean = agg / jnp.maximum(cnt, 1.0)
  h = jnp.maximum(
      jnp.dot(hp_ref[...], ws_ref[...], preferred_element_type=jnp.float32)
      + mean + b_ref[...], 0.0)
  out_ref[...] = jnp.sum(h * wpost_ref[...], axis=1,
                         keepdims=True) + bpost_ref[...]


def _row_spec(shape):
  if len(shape) == 2:
    if shape[0] == NW:
      return pl.BlockSpec((NW, _R), lambda i: (0, i))
    return pl.BlockSpec((_R, shape[1]), lambda i: (i, 0))
  return pl.BlockSpec((shape[0], _R, shape[2]), lambda i: (0, i, 0))


def _full_spec(shape):
  return pl.BlockSpec(shape, lambda i: tuple(0 for _ in shape))


def _tc_call(body, in_shapes, out_shapes, row_in, row_out):
  in_specs = [_row_spec(s) if r else _full_spec(s)
              for s, r in zip(in_shapes, row_in)]
  out_specs = [_row_spec(s) if r else _full_spec(s)
               for s, r in zip(out_shapes, row_out)]
  return pl.pallas_call(
      body,
      grid=(_GRID,),
      in_specs=in_specs,
      out_specs=out_specs[0] if len(out_specs) == 1 else out_specs,
      out_shape=[jax.ShapeDtypeStruct(s, jnp.float32) for s in out_shapes]
      if len(out_shapes) > 1 else jax.ShapeDtypeStruct(out_shapes[0],
                                                       jnp.float32),
  )


def kernel(x, edge_index, W_pre, b_pre, W_self1, W_neigh1, b1,
           W_self2, W_neigh2, b2, W_post, b_post):
  src = edge_index[0]
  dst = edge_index[1]

  pad = E_PAD - E
  src_p = jnp.concatenate(
      [src, jnp.zeros((pad,), jnp.int32)]).reshape(TB, K)
  dst_p = jnp.concatenate(
      [dst, jnp.full((pad,), SINK, jnp.int32)]).reshape(TB, K)

  b_pre2 = b_pre.reshape(1, D)
  b12 = b1.reshape(1, D)
  b22 = b2.reshape(1, D)
  wpost_row = W_post.reshape(1, D)
  bpost2 = b_post.reshape(1, 1)

  tc_pre = _tc_call(
      _tc_pre_body,
      [(N, D), (D, D), (1, D), (D, D)],
      [(N, D), (N, D)],
      [True, False, False, False], [True, True])
  h0, m1 = tc_pre(x, W_pre, b_pre2, W_neigh1)

  agg1, cnt = _sc_aggregate(True)(m1, src_p, dst_p)

  tc_mid = _tc_call(
      _tc_mid_body,
      [(N, D), (NC, TBL, D), (NW, TBL), (D, D), (1, D), (D, D)],
      [(N, D), (N, D)],
      [True, True, True, False, False, False], [True, True])
  h1, m2 = tc_mid(h0, agg1, cnt, W_self1, b12, W_neigh2)

  agg2 = _sc_aggregate(False)(m2, src_p, dst_p)

  tc_post = _tc_call(
      _tc_post_body,
      [(N, D), (NC, TBL, D), (NW, TBL), (D, D), (1, D), (1, D), (1, 1)],
      [(N, 1)],
      [True, True, True, False, False, False, False], [True])
  out = tc_post(h1, agg2, cnt, W_self2, b22, wpost_row, bpost2)
  return jnp.squeeze(out, 1)
